# Initial kernel scaffold; baseline (speedup 1.0000x reference)
import jax
import jax.numpy as jnp
from jax.experimental import pallas as pl


def kernel(x, stem_w, stem_bn_g, stem_bn_b, stem_bn_m, stem_bn_v, b1_w1, b1_bn1_g, b1_bn1_b, b1_bn1_m, b1_bn1_v, b1_w2, b1_bn2_g, b1_bn2_b, b1_bn2_m, b1_bn2_v, b2_w1, b2_bn1_g, b2_bn1_b, b2_bn1_m, b2_bn1_v, b2_w2, b2_bn2_g, b2_bn2_b, b2_bn2_m, b2_bn2_v, b2_wsc, b2_bnsc_g, b2_bnsc_b, b2_bnsc_m, b2_bnsc_v, b3_w1, b3_bn1_g, b3_bn1_b, b3_bn1_m, b3_bn1_v, b3_w2, b3_bn2_g, b3_bn2_b, b3_bn2_m, b3_bn2_v, b3_wsc, b3_bnsc_g, b3_bnsc_b, b3_bnsc_m, b3_bnsc_v, lin_w, lin_b):
    raise NotImplementedError("write your pallas kernel here")


# speedup vs baseline: 1.9705x; 1.9705x over previous
import jax
import jax.numpy as jnp
from jax.experimental import pallas as pl
from jax.experimental.pallas import tpu as pltpu

_IB = 8


def _bn_fold(g, b, m, v, eps=1e-5):
    s = g / jnp.sqrt(v + eps)
    return s, b - m * s


def _pack3x3(w, scale):
    return (w * scale[None, None, None, :]).reshape(9, w.shape[2], w.shape[3])


def _fused_kernel(x_ref, sw, sb, w11, b11, w12, b12,
                  w21, b21, wsc2, bsc2, w22, b22,
                  w31, b31, wsc3, bsc3, w32, b32, lw, lb,
                  o_ref, a1p, t2p, a2p, t3p, *, IB):
    f32 = jnp.float32

    def conv9(slab, w, bias):
        acc = None
        for t in range(9):
            dy, dx = divmod(t, 3)
            d = jnp.dot(slab(dy, dx), w[t], preferred_element_type=f32)
            acc = d if acc is None else acc + d
        return acc + bias[...]

    xv = x_ref[...]

    def xs(dy, dx):
        return xv[:, dy:dy + 32, dx:dx + 32, :].reshape(IB * 1024, 3)

    v0 = jnp.maximum(conv9(xs, sw, sb), 0.0)

    a1p[...] = jnp.zeros(a1p.shape, f32)

    def s1(a):
        def f(dy, dx):
            return a[:, dy:dy + 32, dx:dx + 32, :].reshape(IB * 1024, 64)
        return f

    a1p[:, 1:33, 1:33, :] = v0.reshape(IB, 32, 32, 64)
    v1 = jnp.maximum(conv9(s1(a1p[...]), w11, b11), 0.0)
    a1p[:, 1:33, 1:33, :] = v1.reshape(IB, 32, 32, 64)
    v2 = conv9(s1(a1p[...]), w12, b12)
    a1c = jnp.maximum(v2 + v0, 0.0)
    a1p[:, 1:33, 1:33, :] = a1c.reshape(IB, 32, 32, 64)
    a1v = a1p[...]

    r1 = a1v.reshape(IB, 17, 2, 17, 2, 64)

    def s2(dy, dx):
        i0, pr = (1, 0) if dy == 2 else (0, dy)
        j0, pc = (1, 0) if dx == 2 else (0, dx)
        return r1[:, i0:i0 + 16, pr:pr + 1, j0:j0 + 16, pc:pc + 1, :].reshape(
            IB * 256, 64)

    t2 = jnp.maximum(conv9(s2, w21, b21), 0.0)
    sc2 = jnp.dot(s2(1, 1), wsc2[...], preferred_element_type=f32) + bsc2[...]
    t2p[...] = jnp.zeros(t2p.shape, f32)
    t2p[:, 1:17, 1:17, :] = t2.reshape(IB, 16, 16, 128)
    t2v = t2p[...]

    def s2b(dy, dx):
        return t2v[:, dy:dy + 16, dx:dx + 16, :].reshape(IB * 256, 128)

    a2c = jnp.maximum(conv9(s2b, w22, b22) + sc2, 0.0)
    a2p[...] = jnp.zeros(a2p.shape, f32)
    a2p[:, 1:17, 1:17, :] = a2c.reshape(IB, 16, 16, 128)
    a2v = a2p[...]

    r2 = a2v.reshape(IB, 9, 2, 9, 2, 128)

    def s3(dy, dx):
        i0, pr = (1, 0) if dy == 2 else (0, dy)
        j0, pc = (1, 0) if dx == 2 else (0, dx)
        return r2[:, i0:i0 + 8, pr:pr + 1, j0:j0 + 8, pc:pc + 1, :].reshape(
            IB * 64, 128)

    t3 = jnp.maximum(conv9(s3, w31, b31), 0.0)
    sc3 = jnp.dot(s3(1, 1), wsc3[...], preferred_element_type=f32) + bsc3[...]
    t3p[...] = jnp.zeros(t3p.shape, f32)
    t3p[:, 1:9, 1:9, :] = t3.reshape(IB, 8, 8, 256)
    t3v = t3p[...]

    def s3b(dy, dx):
        return t3v[:, dy:dy + 8, dx:dx + 8, :].reshape(IB * 64, 256)

    a3 = jnp.maximum(conv9(s3b, w32, b32) + sc3, 0.0)

    pooled = jnp.sum(a3.reshape(IB, 64, 256), axis=1)
    o_ref[...] = jnp.dot(pooled, lw[...], preferred_element_type=f32) + lb[...]


def _const_spec(a):
    nd = a.ndim
    return pl.BlockSpec(a.shape, lambda b, nd=nd: (0,) * nd)


def kernel(x, stem_w, stem_bn_g, stem_bn_b, stem_bn_m, stem_bn_v,
           b1_w1, b1_bn1_g, b1_bn1_b, b1_bn1_m, b1_bn1_v,
           b1_w2, b1_bn2_g, b1_bn2_b, b1_bn2_m, b1_bn2_v,
           b2_w1, b2_bn1_g, b2_bn1_b, b2_bn1_m, b2_bn1_v,
           b2_w2, b2_bn2_g, b2_bn2_b, b2_bn2_m, b2_bn2_v,
           b2_wsc, b2_bnsc_g, b2_bnsc_b, b2_bnsc_m, b2_bnsc_v,
           b3_w1, b3_bn1_g, b3_bn1_b, b3_bn1_m, b3_bn1_v,
           b3_w2, b3_bn2_g, b3_bn2_b, b3_bn2_m, b3_bn2_v,
           b3_wsc, b3_bnsc_g, b3_bnsc_b, b3_bnsc_m, b3_bnsc_v,
           lin_w, lin_b):
    B = x.shape[0]
    IB = _IB
    P = stem_w.shape[-1]
    ncls = lin_w.shape[1]

    ss, sb = _bn_fold(stem_bn_g, stem_bn_b, stem_bn_m, stem_bn_v)
    swp = _pack3x3(stem_w, ss)
    s11, bb11 = _bn_fold(b1_bn1_g, b1_bn1_b, b1_bn1_m, b1_bn1_v)
    s12, bb12 = _bn_fold(b1_bn2_g, b1_bn2_b, b1_bn2_m, b1_bn2_v)
    w11 = _pack3x3(b1_w1, s11)
    w12 = _pack3x3(b1_w2, s12)

    s21, bb21 = _bn_fold(b2_bn1_g, b2_bn1_b, b2_bn1_m, b2_bn1_v)
    s22, bb22 = _bn_fold(b2_bn2_g, b2_bn2_b, b2_bn2_m, b2_bn2_v)
    ssc2, bbsc2 = _bn_fold(b2_bnsc_g, b2_bnsc_b, b2_bnsc_m, b2_bnsc_v)
    w21 = _pack3x3(b2_w1, s21)
    w22 = _pack3x3(b2_w2, s22)
    wsc2 = b2_wsc[0, 0] * ssc2[None, :]

    s31, bb31 = _bn_fold(b3_bn1_g, b3_bn1_b, b3_bn1_m, b3_bn1_v)
    s32, bb32 = _bn_fold(b3_bn2_g, b3_bn2_b, b3_bn2_m, b3_bn2_v)
    ssc3, bbsc3 = _bn_fold(b3_bnsc_g, b3_bnsc_b, b3_bnsc_m, b3_bnsc_v)
    w31 = _pack3x3(b3_w1, s31)
    w32 = _pack3x3(b3_w2, s32)
    wsc3 = b3_wsc[0, 0] * ssc3[None, :]

    lw = lin_w * (1.0 / 64.0)
    lb = lin_b.reshape(1, ncls)

    xp = jnp.pad(jnp.transpose(x, (0, 2, 3, 1)).astype(jnp.float32),
                 ((0, 0), (1, 1), (1, 1), (0, 0)))

    consts = [swp, sb.reshape(1, P), w11, bb11.reshape(1, P),
              w12, bb12.reshape(1, P),
              w21, bb21.reshape(1, 2 * P), wsc2, bbsc2.reshape(1, 2 * P),
              w22, bb22.reshape(1, 2 * P),
              w31, bb31.reshape(1, 4 * P), wsc3, bbsc3.reshape(1, 4 * P),
              w32, bb32.reshape(1, 4 * P), lw, lb]

    import functools
    kfn = functools.partial(_fused_kernel, IB=IB)
    out = pl.pallas_call(
        kfn,
        grid=(B // IB,),
        in_specs=([pl.BlockSpec((IB, 34, 34, 3), lambda b: (b, 0, 0, 0))]
                  + [_const_spec(a) for a in consts]),
        out_specs=pl.BlockSpec((IB, ncls), lambda b: (b, 0)),
        out_shape=jax.ShapeDtypeStruct((B, ncls), jnp.float32),
        scratch_shapes=[pltpu.VMEM((IB, 34, 34, P), jnp.float32),
                        pltpu.VMEM((IB, 18, 18, 2 * P), jnp.float32),
                        pltpu.VMEM((IB, 18, 18, 2 * P), jnp.float32),
                        pltpu.VMEM((IB, 10, 10, 4 * P), jnp.float32)],
        compiler_params=pltpu.CompilerParams(
            dimension_semantics=("parallel",)),
    )(xp, *consts)
    return out
